# baseline (speedup 1.0000x reference)
import functools
import math

import jax
import jax.numpy as jnp
from jax.experimental import pallas as pl
from jax.experimental.pallas import tpu as pltpu

H = 2048
NE = 577
NTOK = 577
TOPK = 433
NHEAD = 8
HD = H // NHEAD
B = 4
NP = 640
MP = B * NP
KQKV = 3 * H
SELP = 448

_HIGH = jax.lax.Precision.HIGHEST
_BF = jnp.bfloat16


def _dot_bf(a, b):
    return jax.lax.dot_general(a.astype(_BF), b.astype(_BF),
                               (((1,), (1,)), ((), ())),
                               preferred_element_type=jnp.float32)


def _dot_exact(a, b):
    return jax.lax.dot_general(a, b, (((1,), (0,)), ((), ())),
                               preferred_element_type=jnp.float32,
                               precision=_HIGH)


def _mm_kernel(*refs, nk, act, ln, res):
    a_ref, w_ref, b_ref = refs[0], refs[1], refs[2]
    idx = 3
    res_ref = None
    if res:
        res_ref = refs[idx]
        idx += 1
    if ln:
        lnw_ref, lnb_ref = refs[idx], refs[idx + 1]
        idx += 2
    o_ref = refs[idx]
    acc_ref = refs[idx + 1]

    k = pl.program_id(2)

    @pl.when(k == 0)
    def _():
        acc_ref[...] = jnp.zeros_like(acc_ref)

    acc_ref[...] += jax.lax.dot_general(
        a_ref[...].astype(_BF), w_ref[...], (((1,), (1,)), ((), ())),
        preferred_element_type=jnp.float32)

    @pl.when(k == nk - 1)
    def _():
        r = acc_ref[...] + b_ref[...]
        if act == "relu":
            r = jnp.maximum(r, 0.0)
        elif act == "gelu":
            r = 0.5 * r * (1.0 + jax.lax.erf(r * (1.0 / math.sqrt(2.0))))
        if res:
            r = r + res_ref[...]
        if ln:
            m = jnp.mean(r, axis=1, keepdims=True)
            d = r - m
            v = jnp.mean(d * d, axis=1, keepdims=True)
            r = d / jnp.sqrt(v + 1e-5) * lnw_ref[...] + lnb_ref[...]
        o_ref[...] = r


def _mm(a, w_bf, bias, res=None, lnw=None, lnb=None, act="none",
        bm=512, bk=512, bn=2048):
    m, kdim = a.shape
    n = w_bf.shape[0]
    nm, nn, nk = m // bm, n // bn, kdim // bk
    ln = lnw is not None
    has_res = res is not None
    in_specs = [
        pl.BlockSpec((bm, bk), lambda i, j, k: (i, k)),
        pl.BlockSpec((bn, bk), lambda i, j, k: (j, k)),
        pl.BlockSpec((1, bn), lambda i, j, k: (0, j)),
    ]
    args = [a, w_bf, bias.reshape(1, n)]
    if has_res:
        in_specs.append(pl.BlockSpec((bm, bn), lambda i, j, k: (i, j)))
        args.append(res)
    if ln:
        in_specs.append(pl.BlockSpec((1, bn), lambda i, j, k: (0, j)))
        in_specs.append(pl.BlockSpec((1, bn), lambda i, j, k: (0, j)))
        args += [lnw.reshape(1, n), lnb.reshape(1, n)]
    return pl.pallas_call(
        functools.partial(_mm_kernel, nk=nk, act=act, ln=ln, res=has_res),
        grid=(nm, nn, nk),
        in_specs=in_specs,
        out_specs=pl.BlockSpec((bm, bn), lambda i, j, k: (i, j)),
        out_shape=jax.ShapeDtypeStruct((m, n), jnp.float32),
        scratch_shapes=[pltpu.VMEM((bm, bn), jnp.float32)],
        compiler_params=pltpu.CompilerParams(
            dimension_semantics=("parallel", "parallel", "arbitrary")),
    )(*args)


def _attn_kernel(qkv_ref, mh_ref, mht_ref, o_ref):
    mh = mh_ref[...]
    mht = mht_ref[...]
    scale = 1.0 / math.sqrt(HD)
    qb = [qkv_ref[i, :, 0:H].astype(_BF).astype(jnp.float32)
          for i in range(B)]
    kb = [qkv_ref[i, :, H:2 * H].astype(_BF).astype(jnp.float32)
          for i in range(B)]
    vb = [qkv_ref[i, :, 2 * H:3 * H].astype(_BF).astype(jnp.float32)
          for i in range(B)]
    for i in range(B):
        s = [_dot_exact(qb[i] * kb[j], mh) * scale for j in range(B)]
        mx = jnp.maximum(jnp.maximum(s[0], s[1]), jnp.maximum(s[2], s[3]))
        e = [jnp.exp(s[j] - mx) for j in range(B)]
        den = e[0] + e[1] + e[2] + e[3]
        acc = jnp.zeros(qb[0].shape, jnp.float32)
        for j in range(B):
            aj = (e[j] / den).astype(_BF).astype(jnp.float32)
            ab = _dot_exact(aj, mht)
            acc = acc + (ab * vb[j]).astype(_BF).astype(jnp.float32)
        o_ref[i, :, :] = acc


def _attn(qkv3, mh, mht):
    tb = 128
    return pl.pallas_call(
        _attn_kernel,
        grid=(NP // tb,),
        in_specs=[
            pl.BlockSpec((B, tb, KQKV), lambda t: (0, t, 0)),
            pl.BlockSpec((H, NHEAD), lambda t: (0, 0)),
            pl.BlockSpec((NHEAD, H), lambda t: (0, 0)),
        ],
        out_specs=pl.BlockSpec((B, tb, H), lambda t: (0, t, 0)),
        out_shape=jax.ShapeDtypeStruct((B, NP, H), jnp.float32),
        compiler_params=pltpu.CompilerParams(
            dimension_semantics=("arbitrary",)),
    )(qkv3, mh, mht)


def _rlv_kernel(h_ref, r2_ref, b_ref, vc_ref, o_ref, acc_ref, g_ref,
                *, nm, nk):
    i, k = pl.program_id(0), pl.program_id(1)

    @pl.when(k == 0)
    def _():
        acc_ref[...] = jnp.zeros_like(acc_ref)

    @pl.when((i == 0) & (k == 0))
    def _():
        g_ref[...] = jnp.zeros_like(g_ref)

    acc_ref[...] += jax.lax.dot_general(
        h_ref[...].astype(_BF), r2_ref[...], (((1,), (1,)), ((), ())),
        preferred_element_type=jnp.float32)

    @pl.when(k == nk - 1)
    def _():
        rd = (acc_ref[...] + b_ref[...]).astype(_BF).astype(jnp.float32)
        g_ref[...] += jnp.sum(rd * vc_ref[...], axis=0, keepdims=True)

    @pl.when((i == nm - 1) & (k == nk - 1))
    def _():
        o_ref[...] = g_ref[...]


def _rlv(hr, r2_bf, r2b, vcol, bm=512, bk=512):
    nm, nk = MP // bm, H // bk
    return pl.pallas_call(
        functools.partial(_rlv_kernel, nm=nm, nk=nk),
        grid=(nm, nk),
        in_specs=[
            pl.BlockSpec((bm, bk), lambda i, k: (i, k)),
            pl.BlockSpec((NP, bk), lambda i, k: (0, k)),
            pl.BlockSpec((1, NP), lambda i, k: (0, 0)),
            pl.BlockSpec((bm, 1), lambda i, k: (i, 0)),
        ],
        out_specs=pl.BlockSpec((1, NP), lambda i, k: (0, 0)),
        out_shape=jax.ShapeDtypeStruct((1, NP), jnp.float32),
        scratch_shapes=[pltpu.VMEM((bm, NP), jnp.float32),
                        pltpu.VMEM((1, NP), jnp.float32)],
        compiler_params=pltpu.CompilerParams(
            dimension_semantics=("arbitrary", "arbitrary")),
    )(hr, r2_bf, r2b.reshape(1, NP), vcol)


def _route_kernel(rlv_ref, nw_ref, nb_ref, rl_ref, sw_ref):
    rlv = rlv_ref[...]
    t_iota = jax.lax.broadcasted_iota(jnp.int32, (1, NP), 1)
    mask = t_iota < NE
    cnt = float(NE)
    mean = jnp.sum(jnp.where(mask, rlv, 0.0)) / cnt
    d = jnp.where(mask, rlv - mean, 0.0)
    var = jnp.sum(d * d) / cnt
    rln = d / jnp.sqrt(var + 1e-5) * nw_ref[...] + nb_ref[...]
    rl_ref[...] = rln

    neg = jnp.float32(-jnp.inf)
    vrow = jnp.where(mask, rln, neg)
    eye = (jax.lax.broadcasted_iota(jnp.int32, (NP, NP), 0) ==
           jax.lax.broadcasted_iota(jnp.int32, (NP, NP), 1)).astype(jnp.float32)
    u_iota = jax.lax.broadcasted_iota(jnp.int32, (NP, 1), 0)
    vcolT = jax.lax.dot_general(eye, rln, (((1,), (1,)), ((), ())),
                                preferred_element_type=jnp.float32,
                                precision=jax.lax.Precision.HIGHEST)
    vcol = jnp.where(u_iota < NE, vcolT, neg)
    lt = (jax.lax.broadcasted_iota(jnp.int32, (NP, NP), 0) <
          jax.lax.broadcasted_iota(jnp.int32, (NP, NP), 1))
    gt = (vcol > vrow).astype(jnp.float32)
    eqlt = ((vcol == vrow) & lt).astype(jnp.float32)
    rank = jnp.sum(gt + eqlt, axis=0, keepdims=True)
    sel = ((rank < float(TOPK)) & mask).astype(jnp.float32)
    selcol = jax.lax.dot_general(eye, sel, (((1,), (1,)), ((), ())),
                                 preferred_element_type=jnp.float32,
                                 precision=jax.lax.Precision.HIGHEST)
    cume = jnp.sum(selcol * lt.astype(jnp.float32), axis=0, keepdims=True)
    vsel = jnp.where(sel > 0.5, rln, 0.0)
    w = vsel / jnp.sum(vsel)
    jrow = jax.lax.broadcasted_iota(jnp.int32, (SELP, NP), 0)
    cume_i = cume.astype(jnp.int32)
    sw_ref[...] = jnp.where(jrow == cume_i, 1.0, 0.0) * w


def _route(rlv, nwp, nbp):
    return pl.pallas_call(
        _route_kernel,
        in_specs=[pl.BlockSpec((1, NP), lambda: (0, 0)),
                  pl.BlockSpec((1, NP), lambda: (0, 0)),
                  pl.BlockSpec((1, NP), lambda: (0, 0))],
        out_specs=[pl.BlockSpec((1, NP), lambda: (0, 0)),
                   pl.BlockSpec((SELP, NP), lambda: (0, 0))],
        out_shape=[jax.ShapeDtypeStruct((1, NP), jnp.float32),
                   jax.ShapeDtypeStruct((SELP, NP), jnp.float32)],
    )(rlv, nwp, nbp)


def _final_kernel(sw_ref, x_ref, o_ref):
    o_ref[0] = jax.lax.dot_general(
        sw_ref[...], x_ref[0], (((1,), (0,)), ((), ())),
        preferred_element_type=jnp.float32, precision=_HIGH)


def _final(sw, x2r):
    return pl.pallas_call(
        _final_kernel,
        grid=(B,),
        in_specs=[pl.BlockSpec((SELP, NP), lambda b: (0, 0)),
                  pl.BlockSpec((1, NP, H), lambda b: (b, 0, 0))],
        out_specs=pl.BlockSpec((1, SELP, H), lambda b: (b, 0, 0)),
        out_shape=jax.ShapeDtypeStruct((B, SELP, H), jnp.float32),
        compiler_params=pltpu.CompilerParams(
            dimension_semantics=("arbitrary",)),
    )(sw, x2r)


def kernel(hidden_states, text_hidden_states, label_hidden_states,
           label_mask, params):
    p = params
    x = jnp.concatenate([hidden_states, text_hidden_states], axis=1)
    x = jnp.pad(x, ((0, 0), (0, NP - NTOK), (0, 0)))
    xf = x.reshape(MP, H)

    dh = jnp.arange(H, dtype=jnp.int32) // HD
    mh = (dh[:, None] == jnp.arange(NHEAD, dtype=jnp.int32)[None, :]
          ).astype(jnp.float32)
    mht = mh.T

    w_in = p['in_proj_w'].astype(_BF)
    w_out = p['out_proj_w'].astype(_BF)
    w_l1 = p['l1_w'].astype(_BF)
    w_l2 = p['l2_w'].astype(_BF)
    w_r1 = p['r1_w'].astype(_BF)

    qkv = _mm(xf, w_in, p['in_proj_b'])
    o = _attn(qkv.reshape(B, NP, KQKV), mh, mht)
    x1 = _mm(o.reshape(MP, H), w_out, p['out_proj_b'],
             res=xf, lnw=p['ln1_w'], lnb=p['ln1_b'])
    h1 = _mm(x1, w_l1, p['l1_b'], act="relu")
    x2 = _mm(h1, w_l2, p['l2_b'],
             res=x1, lnw=p['ln2_w'], lnb=p['ln2_b'])
    hr = _mm(x2, w_r1, p['r1_b'], act="gelu")

    voter_bf = p['voter'][:, 0].astype(_BF).astype(jnp.float32)
    vp = jnp.pad(voter_bf, (0, NP - NTOK)) * 0.25
    vcol = jnp.tile(vp, (B,)).reshape(MP, 1)

    r2_bf = jnp.pad(p['r2_w'], ((0, NP - NE), (0, 0))).astype(_BF)
    r2b = jnp.pad(p['r2_b'], (0, NP - NE))
    rlv = _rlv(hr, r2_bf, r2b, vcol)

    nwp = jnp.pad(p['norm_w'].reshape(1, NE), ((0, 0), (0, NP - NE)))
    nbp = jnp.pad(p['norm_b'].reshape(1, NE), ((0, 0), (0, NP - NE)))
    rl_p, sw = _route(rlv, nwp, nbp)

    fin = _final(sw, x2.reshape(B, NP, H))
    return fin[:, :TOPK, :], rl_p[:, :NE]


# speedup vs baseline: 1.2214x; 1.2214x over previous
import functools
import math

import jax
import jax.numpy as jnp
from jax.experimental import pallas as pl
from jax.experimental.pallas import tpu as pltpu

H = 2048
NE = 577
NTOK = 577
TOPK = 433
NHEAD = 8
HD = H // NHEAD
B = 4
NP = 640
MP = B * NP
KQKV = 3 * H
SELP = 448

_HIGH = jax.lax.Precision.HIGHEST
_BF = jnp.bfloat16


def _mm_kernel(*refs, nk, act, ln, res):
    a_ref, w_ref, b_ref = refs[0], refs[1], refs[2]
    idx = 3
    res_ref = None
    if res:
        res_ref = refs[idx]
        idx += 1
    if ln:
        lnw_ref, lnb_ref = refs[idx], refs[idx + 1]
        idx += 2
    o_ref = refs[idx]
    acc_ref = refs[idx + 1]

    k = pl.program_id(2)

    @pl.when(k == 0)
    def _():
        acc_ref[...] = jnp.zeros_like(acc_ref)

    acc_ref[...] += jax.lax.dot_general(
        a_ref[...].astype(_BF), w_ref[...], (((1,), (1,)), ((), ())),
        preferred_element_type=jnp.float32)

    @pl.when(k == nk - 1)
    def _():
        r = acc_ref[...] + b_ref[...]
        if act == "relu":
            r = jnp.maximum(r, 0.0)
        elif act == "gelu":
            r = 0.5 * r * (1.0 + jax.lax.erf(r * (1.0 / math.sqrt(2.0))))
        if res:
            r = r + res_ref[...]
        if ln:
            m = jnp.mean(r, axis=1, keepdims=True)
            d = r - m
            v = jnp.mean(d * d, axis=1, keepdims=True)
            r = d / jnp.sqrt(v + 1e-5) * lnw_ref[...] + lnb_ref[...]
        o_ref[...] = r


def _mm(a, w_bf, bias, res=None, lnw=None, lnb=None, act="none",
        bm=512, bk=512, bn=2048):
    m, kdim = a.shape
    n = w_bf.shape[0]
    nm, nn, nk = m // bm, n // bn, kdim // bk
    ln = lnw is not None
    has_res = res is not None
    in_specs = [
        pl.BlockSpec((bm, bk), lambda i, j, k: (i, k)),
        pl.BlockSpec((bn, bk), lambda i, j, k: (j, k)),
        pl.BlockSpec((1, bn), lambda i, j, k: (0, j)),
    ]
    args = [a, w_bf, bias.reshape(1, n)]
    if has_res:
        in_specs.append(pl.BlockSpec((bm, bn), lambda i, j, k: (i, j)))
        args.append(res)
    if ln:
        in_specs.append(pl.BlockSpec((1, bn), lambda i, j, k: (0, j)))
        in_specs.append(pl.BlockSpec((1, bn), lambda i, j, k: (0, j)))
        args += [lnw.reshape(1, n), lnb.reshape(1, n)]
    return pl.pallas_call(
        functools.partial(_mm_kernel, nk=nk, act=act, ln=ln, res=has_res),
        grid=(nm, nn, nk),
        in_specs=in_specs,
        out_specs=pl.BlockSpec((bm, bn), lambda i, j, k: (i, j)),
        out_shape=jax.ShapeDtypeStruct((m, n), jnp.float32),
        scratch_shapes=[pltpu.VMEM((bm, bn), jnp.float32)],
        compiler_params=pltpu.CompilerParams(
            dimension_semantics=("parallel", "parallel", "arbitrary")),
    )(*args)


def _attn_kernel(qkv_ref, o_ref):
    scale = 1.0 / math.sqrt(HD)
    tb = qkv_ref.shape[1]
    qb = [qkv_ref[i, :, 0:H].astype(_BF).astype(jnp.float32)
          for i in range(B)]
    kb = [qkv_ref[i, :, H:2 * H].astype(_BF).astype(jnp.float32)
          for i in range(B)]
    vb = [qkv_ref[i, :, 2 * H:3 * H].astype(_BF).astype(jnp.float32)
          for i in range(B)]
    for i in range(B):
        s = [(qb[i] * kb[j]).reshape(tb, NHEAD, HD).sum(axis=-1) * scale
             for j in range(B)]
        mx = jnp.maximum(jnp.maximum(s[0], s[1]), jnp.maximum(s[2], s[3]))
        e = [jnp.exp(s[j] - mx) for j in range(B)]
        den = e[0] + e[1] + e[2] + e[3]
        acc = jnp.zeros((tb, H), jnp.float32)
        for j in range(B):
            aj = (e[j] / den).astype(_BF).astype(jnp.float32)
            ab = jnp.broadcast_to(aj[:, :, None], (tb, NHEAD, HD)
                                  ).reshape(tb, H)
            acc = acc + (ab * vb[j]).astype(_BF).astype(jnp.float32)
        o_ref[i, :, :] = acc


def _attn(qkv3):
    tb = 128
    return pl.pallas_call(
        _attn_kernel,
        grid=(NP // tb,),
        in_specs=[
            pl.BlockSpec((B, tb, KQKV), lambda t: (0, t, 0)),
        ],
        out_specs=pl.BlockSpec((B, tb, H), lambda t: (0, t, 0)),
        out_shape=jax.ShapeDtypeStruct((B, NP, H), jnp.float32),
        compiler_params=pltpu.CompilerParams(
            dimension_semantics=("arbitrary",)),
    )(qkv3)


def _rlv_kernel(h_ref, r2_ref, b_ref, vc_ref, o_ref, acc_ref, g_ref,
                *, nm, nk):
    i, k = pl.program_id(0), pl.program_id(1)

    @pl.when(k == 0)
    def _():
        acc_ref[...] = jnp.zeros_like(acc_ref)

    @pl.when((i == 0) & (k == 0))
    def _():
        g_ref[...] = jnp.zeros_like(g_ref)

    acc_ref[...] += jax.lax.dot_general(
        h_ref[...].astype(_BF), r2_ref[...], (((1,), (1,)), ((), ())),
        preferred_element_type=jnp.float32)

    @pl.when(k == nk - 1)
    def _():
        rd = (acc_ref[...] + b_ref[...]).astype(_BF).astype(jnp.float32)
        g_ref[...] += jnp.sum(rd * vc_ref[...], axis=0, keepdims=True)

    @pl.when((i == nm - 1) & (k == nk - 1))
    def _():
        o_ref[...] = g_ref[...]


def _rlv(hr, r2_bf, r2b, vcol, bm=512, bk=512):
    nm, nk = MP // bm, H // bk
    return pl.pallas_call(
        functools.partial(_rlv_kernel, nm=nm, nk=nk),
        grid=(nm, nk),
        in_specs=[
            pl.BlockSpec((bm, bk), lambda i, k: (i, k)),
            pl.BlockSpec((NP, bk), lambda i, k: (0, k)),
            pl.BlockSpec((1, NP), lambda i, k: (0, 0)),
            pl.BlockSpec((bm, 1), lambda i, k: (i, 0)),
        ],
        out_specs=pl.BlockSpec((1, NP), lambda i, k: (0, 0)),
        out_shape=jax.ShapeDtypeStruct((1, NP), jnp.float32),
        scratch_shapes=[pltpu.VMEM((bm, NP), jnp.float32),
                        pltpu.VMEM((1, NP), jnp.float32)],
        compiler_params=pltpu.CompilerParams(
            dimension_semantics=("arbitrary", "arbitrary")),
    )(hr, r2_bf, r2b.reshape(1, NP), vcol)


def _route_kernel(rlv_ref, nw_ref, nb_ref, rl_ref, sw_ref):
    rlv = rlv_ref[...]
    t_iota = jax.lax.broadcasted_iota(jnp.int32, (1, NP), 1)
    mask = t_iota < NE
    cnt = float(NE)
    mean = jnp.sum(jnp.where(mask, rlv, 0.0)) / cnt
    d = jnp.where(mask, rlv - mean, 0.0)
    var = jnp.sum(d * d) / cnt
    rln = d / jnp.sqrt(var + 1e-5) * nw_ref[...] + nb_ref[...]
    rl_ref[...] = rln

    neg = jnp.float32(-jnp.inf)
    vrow = jnp.where(mask, rln, neg)
    eye = (jax.lax.broadcasted_iota(jnp.int32, (NP, NP), 0) ==
           jax.lax.broadcasted_iota(jnp.int32, (NP, NP), 1)).astype(jnp.float32)
    u_iota = jax.lax.broadcasted_iota(jnp.int32, (NP, 1), 0)
    vcolT = jax.lax.dot_general(eye, rln, (((1,), (1,)), ((), ())),
                                preferred_element_type=jnp.float32,
                                precision=jax.lax.Precision.HIGHEST)
    vcol = jnp.where(u_iota < NE, vcolT, neg)
    lt = (jax.lax.broadcasted_iota(jnp.int32, (NP, NP), 0) <
          jax.lax.broadcasted_iota(jnp.int32, (NP, NP), 1))
    gt = (vcol > vrow).astype(jnp.float32)
    eqlt = ((vcol == vrow) & lt).astype(jnp.float32)
    rank = jnp.sum(gt + eqlt, axis=0, keepdims=True)
    sel = ((rank < float(TOPK)) & mask).astype(jnp.float32)
    selcol = jax.lax.dot_general(eye, sel, (((1,), (1,)), ((), ())),
                                 preferred_element_type=jnp.float32,
                                 precision=jax.lax.Precision.HIGHEST)
    cume = jnp.sum(selcol * lt.astype(jnp.float32), axis=0, keepdims=True)
    vsel = jnp.where(sel > 0.5, rln, 0.0)
    w = vsel / jnp.sum(vsel)
    jrow = jax.lax.broadcasted_iota(jnp.int32, (SELP, NP), 0)
    cume_i = cume.astype(jnp.int32)
    sw_ref[...] = jnp.where(jrow == cume_i, 1.0, 0.0) * w


def _route(rlv, nwp, nbp):
    return pl.pallas_call(
        _route_kernel,
        in_specs=[pl.BlockSpec((1, NP), lambda: (0, 0)),
                  pl.BlockSpec((1, NP), lambda: (0, 0)),
                  pl.BlockSpec((1, NP), lambda: (0, 0))],
        out_specs=[pl.BlockSpec((1, NP), lambda: (0, 0)),
                   pl.BlockSpec((SELP, NP), lambda: (0, 0))],
        out_shape=[jax.ShapeDtypeStruct((1, NP), jnp.float32),
                   jax.ShapeDtypeStruct((SELP, NP), jnp.float32)],
    )(rlv, nwp, nbp)


def _final_kernel(sw_ref, x_ref, o_ref):
    sw = sw_ref[...]
    x = x_ref[0]
    swh = sw.astype(_BF)
    swl = (sw - swh.astype(jnp.float32)).astype(_BF)
    xh = x.astype(_BF)
    xl = (x - xh.astype(jnp.float32)).astype(_BF)

    def d(a, b):
        return jax.lax.dot_general(a, b, (((1,), (0,)), ((), ())),
                                   preferred_element_type=jnp.float32)

    o_ref[0] = d(swh, xh) + (d(swh, xl) + d(swl, xh))


def _final(sw, x2r):
    return pl.pallas_call(
        _final_kernel,
        grid=(B,),
        in_specs=[pl.BlockSpec((SELP, NP), lambda b: (0, 0)),
                  pl.BlockSpec((1, NP, H), lambda b: (b, 0, 0))],
        out_specs=pl.BlockSpec((1, SELP, H), lambda b: (b, 0, 0)),
        out_shape=jax.ShapeDtypeStruct((B, SELP, H), jnp.float32),
        compiler_params=pltpu.CompilerParams(
            dimension_semantics=("arbitrary",)),
    )(sw, x2r)


def kernel(hidden_states, text_hidden_states, label_hidden_states,
           label_mask, params):
    p = params
    x = jnp.concatenate([hidden_states, text_hidden_states], axis=1)
    x = jnp.pad(x, ((0, 0), (0, NP - NTOK), (0, 0)))
    xf = x.reshape(MP, H)

    w_in = p['in_proj_w'].astype(_BF)
    w_out = p['out_proj_w'].astype(_BF)
    w_l1 = p['l1_w'].astype(_BF)
    w_l2 = p['l2_w'].astype(_BF)
    w_r1 = p['r1_w'].astype(_BF)

    qkv = _mm(xf, w_in, p['in_proj_b'])
    o = _attn(qkv.reshape(B, NP, KQKV))
    x1 = _mm(o.reshape(MP, H), w_out, p['out_proj_b'],
             res=xf, lnw=p['ln1_w'], lnb=p['ln1_b'])
    h1 = _mm(x1, w_l1, p['l1_b'], act="relu")
    x2 = _mm(h1, w_l2, p['l2_b'],
             res=x1, lnw=p['ln2_w'], lnb=p['ln2_b'])
    hr = _mm(x2, w_r1, p['r1_b'], act="gelu")

    voter_bf = p['voter'][:, 0].astype(_BF).astype(jnp.float32)
    vp = jnp.pad(voter_bf, (0, NP - NTOK)) * 0.25
    vcol = jnp.tile(vp, (B,)).reshape(MP, 1)

    r2_bf = jnp.pad(p['r2_w'], ((0, NP - NE), (0, 0))).astype(_BF)
    r2b = jnp.pad(p['r2_b'], (0, NP - NE))
    rlv = _rlv(hr, r2_bf, r2b, vcol)

    nwp = jnp.pad(p['norm_w'].reshape(1, NE), ((0, 0), (0, NP - NE)))
    nbp = jnp.pad(p['norm_b'].reshape(1, NE), ((0, 0), (0, NP - NE)))
    rl_p, sw = _route(rlv, nwp, nbp)

    fin = _final(sw, x2.reshape(B, NP, H))
    return fin[:, :TOPK, :], rl_p[:, :NE]


# speedup vs baseline: 1.5597x; 1.2770x over previous
import functools
import math

import jax
import jax.numpy as jnp
from jax.experimental import pallas as pl
from jax.experimental.pallas import tpu as pltpu

H = 2048
NE = 577
NTOK = 577
TOPK = 433
NHEAD = 8
HD = H // NHEAD
B = 4
NP = 640
MP = B * NP
KQKV = 3 * H
SELP = 448

_BF = jnp.bfloat16
_F32 = jnp.float32


def _mm_kernel(*refs, nk, act, ln, res, out_dtype):
    a_ref, w_ref, b_ref = refs[0], refs[1], refs[2]
    idx = 3
    res_ref = None
    if res:
        res_ref = refs[idx]
        idx += 1
    if ln:
        lnw_ref, lnb_ref = refs[idx], refs[idx + 1]
        idx += 2
    o_ref = refs[idx]
    acc_ref = refs[idx + 1]

    k = pl.program_id(2)

    @pl.when(k == 0)
    def _():
        acc_ref[...] = jnp.zeros_like(acc_ref)

    acc_ref[...] += jax.lax.dot_general(
        a_ref[...].astype(_BF), w_ref[...].astype(_BF),
        (((1,), (1,)), ((), ())), preferred_element_type=_F32)

    @pl.when(k == nk - 1)
    def _():
        r = acc_ref[...] + b_ref[...]
        if act == "relu":
            r = jnp.maximum(r, 0.0)
        elif act == "gelu":
            r = 0.5 * r * (1.0 + jax.lax.erf(r * (1.0 / math.sqrt(2.0))))
        if res:
            r = r + res_ref[...]
        if ln:
            m = jnp.mean(r, axis=1, keepdims=True)
            d = r - m
            v = jnp.mean(d * d, axis=1, keepdims=True)
            r = d / jnp.sqrt(v + 1e-5) * lnw_ref[...] + lnb_ref[...]
        o_ref[...] = r.astype(out_dtype)


def _mm(a, w, bias, res=None, lnw=None, lnb=None, act="none",
        bm=512, bk=512, bn=2048, out_dtype=_F32):
    m, kdim = a.shape
    n = w.shape[0]
    nm, nn, nk = m // bm, n // bn, kdim // bk
    ln = lnw is not None
    has_res = res is not None
    in_specs = [
        pl.BlockSpec((bm, bk), lambda i, j, k: (i, k)),
        pl.BlockSpec((bn, bk), lambda i, j, k: (j, k)),
        pl.BlockSpec((1, bn), lambda i, j, k: (0, j)),
    ]
    args = [a, w, bias.reshape(1, n)]
    if has_res:
        in_specs.append(pl.BlockSpec((bm, bn), lambda i, j, k: (i, j)))
        args.append(res)
    if ln:
        in_specs.append(pl.BlockSpec((1, bn), lambda i, j, k: (0, j)))
        in_specs.append(pl.BlockSpec((1, bn), lambda i, j, k: (0, j)))
        args += [lnw.reshape(1, n), lnb.reshape(1, n)]
    return pl.pallas_call(
        functools.partial(_mm_kernel, nk=nk, act=act, ln=ln, res=has_res,
                          out_dtype=out_dtype),
        grid=(nm, nn, nk),
        in_specs=in_specs,
        out_specs=pl.BlockSpec((bm, bn), lambda i, j, k: (i, j)),
        out_shape=jax.ShapeDtypeStruct((m, n), out_dtype),
        scratch_shapes=[pltpu.VMEM((bm, bn), _F32)],
        compiler_params=pltpu.CompilerParams(
            dimension_semantics=("parallel", "parallel", "arbitrary")),
    )(*args)


def _attn_kernel(qkv_ref, o_ref):
    scale = 1.0 / math.sqrt(HD)
    tb = qkv_ref.shape[1]
    qb = [qkv_ref[i, :, 0:H].astype(_F32) for i in range(B)]
    kb = [qkv_ref[i, :, H:2 * H].astype(_F32) for i in range(B)]
    vb = [qkv_ref[i, :, 2 * H:3 * H].astype(_F32) for i in range(B)]
    for i in range(B):
        s = [(qb[i] * kb[j]).reshape(tb, NHEAD, HD).sum(axis=-1) * scale
             for j in range(B)]
        mx = jnp.maximum(jnp.maximum(s[0], s[1]), jnp.maximum(s[2], s[3]))
        e = [jnp.exp(s[j] - mx) for j in range(B)]
        den = e[0] + e[1] + e[2] + e[3]
        acc = jnp.zeros((tb, H), _F32)
        for j in range(B):
            aj = (e[j] / den).astype(_BF).astype(_F32)
            ab = jnp.broadcast_to(aj[:, :, None], (tb, NHEAD, HD)
                                  ).reshape(tb, H)
            acc = acc + (ab * vb[j]).astype(_BF).astype(_F32)
        o_ref[i, :, :] = acc.astype(_BF)


def _attn(qkv3):
    tb = 128
    return pl.pallas_call(
        _attn_kernel,
        grid=(NP // tb,),
        in_specs=[
            pl.BlockSpec((B, tb, KQKV), lambda t: (0, t, 0)),
        ],
        out_specs=pl.BlockSpec((B, tb, H), lambda t: (0, t, 0)),
        out_shape=jax.ShapeDtypeStruct((B, NP, H), _BF),
        compiler_params=pltpu.CompilerParams(
            dimension_semantics=("arbitrary",)),
    )(qkv3)


def _rlv_kernel(h_ref, r2_ref, b_ref, vc_ref, o_ref, acc_ref, g_ref,
                *, nm, nk):
    i, k = pl.program_id(0), pl.program_id(1)

    @pl.when(k == 0)
    def _():
        acc_ref[...] = jnp.zeros_like(acc_ref)

    @pl.when((i == 0) & (k == 0))
    def _():
        g_ref[...] = jnp.zeros_like(g_ref)

    acc_ref[...] += jax.lax.dot_general(
        h_ref[...].astype(_BF), r2_ref[...].astype(_BF),
        (((1,), (1,)), ((), ())), preferred_element_type=_F32)

    @pl.when(k == nk - 1)
    def _():
        rd = (acc_ref[...] + b_ref[...]).astype(_BF).astype(_F32)
        g_ref[...] += jnp.sum(rd * vc_ref[...], axis=0, keepdims=True)

    @pl.when((i == nm - 1) & (k == nk - 1))
    def _():
        o_ref[...] = g_ref[...]


def _rlv(hr, r2p, r2b, vcol, bm=1280, bk=512):
    nm, nk = MP // bm, H // bk
    return pl.pallas_call(
        functools.partial(_rlv_kernel, nm=nm, nk=nk),
        grid=(nm, nk),
        in_specs=[
            pl.BlockSpec((bm, bk), lambda i, k: (i, k)),
            pl.BlockSpec((NP, bk), lambda i, k: (0, k)),
            pl.BlockSpec((1, NP), lambda i, k: (0, 0)),
            pl.BlockSpec((bm, 1), lambda i, k: (i, 0)),
        ],
        out_specs=pl.BlockSpec((1, NP), lambda i, k: (0, 0)),
        out_shape=jax.ShapeDtypeStruct((1, NP), _F32),
        scratch_shapes=[pltpu.VMEM((bm, NP), _F32),
                        pltpu.VMEM((1, NP), _F32)],
        compiler_params=pltpu.CompilerParams(
            dimension_semantics=("arbitrary", "arbitrary")),
    )(hr, r2p, r2b.reshape(1, NP), vcol)


def _route_kernel(rlv_ref, nw_ref, nb_ref, rl_ref, sw_ref):
    rlv = rlv_ref[...]
    t_iota = jax.lax.broadcasted_iota(jnp.int32, (1, NP), 1)
    mask = t_iota < NE
    cnt = float(NE)
    mean = jnp.sum(jnp.where(mask, rlv, 0.0)) / cnt
    d = jnp.where(mask, rlv - mean, 0.0)
    var = jnp.sum(d * d) / cnt
    rln = d / jnp.sqrt(var + 1e-5) * nw_ref[...] + nb_ref[...]
    rl_ref[...] = rln

    neg = jnp.float32(-jnp.inf)
    vrow = jnp.where(mask, rln, neg)
    eye = (jax.lax.broadcasted_iota(jnp.int32, (NP, NP), 0) ==
           jax.lax.broadcasted_iota(jnp.int32, (NP, NP), 1)).astype(_F32)
    u_iota = jax.lax.broadcasted_iota(jnp.int32, (NP, 1), 0)
    vcolT = jax.lax.dot_general(eye, rln, (((1,), (1,)), ((), ())),
                                preferred_element_type=_F32,
                                precision=jax.lax.Precision.HIGHEST)
    vcol = jnp.where(u_iota < NE, vcolT, neg)
    lt = (jax.lax.broadcasted_iota(jnp.int32, (NP, NP), 0) <
          jax.lax.broadcasted_iota(jnp.int32, (NP, NP), 1))
    gt = (vcol > vrow).astype(_F32)
    eqlt = ((vcol == vrow) & lt).astype(_F32)
    rank = jnp.sum(gt + eqlt, axis=0, keepdims=True)
    sel = ((rank < float(TOPK)) & mask).astype(_F32)
    selcol = jax.lax.dot_general(eye, sel, (((1,), (1,)), ((), ())),
                                 preferred_element_type=_F32,
                                 precision=jax.lax.Precision.HIGHEST)
    cume = jnp.sum(selcol * lt.astype(_F32), axis=0, keepdims=True)
    vsel = jnp.where(sel > 0.5, rln, 0.0)
    w = vsel / jnp.sum(vsel)
    jrow = jax.lax.broadcasted_iota(jnp.int32, (SELP, NP), 0)
    cume_i = cume.astype(jnp.int32)
    sw_ref[...] = jnp.where(jrow == cume_i, 1.0, 0.0) * w


def _route(rlv, nwp, nbp):
    return pl.pallas_call(
        _route_kernel,
        in_specs=[pl.BlockSpec((1, NP), lambda: (0, 0)),
                  pl.BlockSpec((1, NP), lambda: (0, 0)),
                  pl.BlockSpec((1, NP), lambda: (0, 0))],
        out_specs=[pl.BlockSpec((1, NP), lambda: (0, 0)),
                   pl.BlockSpec((SELP, NP), lambda: (0, 0))],
        out_shape=[jax.ShapeDtypeStruct((1, NP), _F32),
                   jax.ShapeDtypeStruct((SELP, NP), _F32)],
    )(rlv, nwp, nbp)


def _final_kernel(sw_ref, x_ref, o_ref):
    sw = sw_ref[...]
    x = x_ref[0]
    swh = sw.astype(_BF)
    swl = (sw - swh.astype(_F32)).astype(_BF)
    xh = x.astype(_BF)
    xl = (x - xh.astype(_F32)).astype(_BF)

    def d(a, b):
        return jax.lax.dot_general(a, b, (((1,), (0,)), ((), ())),
                                   preferred_element_type=_F32)

    o_ref[0] = d(swh, xh) + (d(swh, xl) + d(swl, xh))


def _final(sw, x2r):
    return pl.pallas_call(
        _final_kernel,
        grid=(B,),
        in_specs=[pl.BlockSpec((SELP, NP), lambda b: (0, 0)),
                  pl.BlockSpec((1, NP, H), lambda b: (b, 0, 0))],
        out_specs=pl.BlockSpec((1, SELP, H), lambda b: (b, 0, 0)),
        out_shape=jax.ShapeDtypeStruct((B, SELP, H), _F32),
        compiler_params=pltpu.CompilerParams(
            dimension_semantics=("arbitrary",)),
    )(sw, x2r)


def kernel(hidden_states, text_hidden_states, label_hidden_states,
           label_mask, params):
    p = params
    x = jnp.concatenate([hidden_states, text_hidden_states], axis=1)
    x = jnp.pad(x, ((0, 0), (0, NP - NTOK), (0, 0)))
    xf = x.reshape(MP, H)

    qkv = _mm(xf, p['in_proj_w'], p['in_proj_b'],
              bm=1280, out_dtype=_BF)
    o = _attn(qkv.reshape(B, NP, KQKV))
    x1 = _mm(o.reshape(MP, H), p['out_proj_w'], p['out_proj_b'],
             res=xf, lnw=p['ln1_w'], lnb=p['ln1_b'], bm=640)
    h1 = _mm(x1, p['l1_w'], p['l1_b'], act="relu",
             bm=1280, out_dtype=_BF)
    x2 = _mm(h1, p['l2_w'], p['l2_b'],
             res=x1, lnw=p['ln2_w'], lnb=p['ln2_b'], bm=640)
    hr = _mm(x2, p['r1_w'], p['r1_b'], act="gelu",
             bm=1280, out_dtype=_BF)

    voter_bf = p['voter'][:, 0].astype(_BF).astype(_F32)
    vp = jnp.pad(voter_bf, (0, NP - NTOK)) * 0.25
    vcol = jnp.tile(vp, (B,)).reshape(MP, 1)

    r2p = jnp.pad(p['r2_w'], ((0, NP - NE), (0, 0)))
    r2b = jnp.pad(p['r2_b'], (0, NP - NE))
    rlv = _rlv(hr, r2p, r2b, vcol)

    nwp = jnp.pad(p['norm_w'].reshape(1, NE), ((0, 0), (0, NP - NE)))
    nbp = jnp.pad(p['norm_b'].reshape(1, NE), ((0, 0), (0, NP - NE)))
    rl_p, sw = _route(rlv, nwp, nbp)

    fin = _final(sw, x2.reshape(B, NP, H))
    return fin[:, :TOPK, :], rl_p[:, :NE]


# speedup vs baseline: 1.6900x; 1.0835x over previous
import functools
import math

import jax
import jax.numpy as jnp
from jax.experimental import pallas as pl
from jax.experimental.pallas import tpu as pltpu

H = 2048
NE = 577
NTOK = 577
TOPK = 433
NHEAD = 8
HD = H // NHEAD
B = 4
NP = 640
MP = B * NP
KQKV = 3 * H
SELP = 448

_BF = jnp.bfloat16
_F32 = jnp.float32


def _mm_kernel(*refs, act, ln, res, out_dtype):
    a_ref, w_ref, b_ref = refs[0], refs[1], refs[2]
    idx = 3
    res_ref = None
    if res:
        res_ref = refs[idx]
        idx += 1
    if ln:
        lnw_ref, lnb_ref = refs[idx], refs[idx + 1]
        idx += 2
    o_ref = refs[idx]

    r = jax.lax.dot_general(
        a_ref[...].astype(_BF), w_ref[...].astype(_BF),
        (((1,), (1,)), ((), ())), preferred_element_type=_F32)
    r = r + b_ref[...]
    if act == "relu":
        r = jnp.maximum(r, 0.0)
    elif act == "gelu":
        r = 0.5 * r * (1.0 + jax.lax.erf(r * (1.0 / math.sqrt(2.0))))
    if res:
        r = r + res_ref[...]
    if ln:
        m = jnp.mean(r, axis=1, keepdims=True)
        d = r - m
        v = jnp.mean(d * d, axis=1, keepdims=True)
        r = d / jnp.sqrt(v + 1e-5) * lnw_ref[...] + lnb_ref[...]
    o_ref[...] = r.astype(out_dtype)


def _mm(a, w, bias, res=None, lnw=None, lnb=None, act="none",
        bm=512, bn=2048, out_dtype=_F32):
    m, kdim = a.shape
    n = w.shape[0]
    nm, nn = m // bm, n // bn
    ln = lnw is not None
    has_res = res is not None
    in_specs = [
        pl.BlockSpec((bm, kdim), lambda j, i: (i, 0)),
        pl.BlockSpec((bn, kdim), lambda j, i: (j, 0)),
        pl.BlockSpec((1, bn), lambda j, i: (0, j)),
    ]
    args = [a, w, bias.reshape(1, n)]
    if has_res:
        in_specs.append(pl.BlockSpec((bm, bn), lambda j, i: (i, j)))
        args.append(res)
    if ln:
        in_specs.append(pl.BlockSpec((1, bn), lambda j, i: (0, j)))
        in_specs.append(pl.BlockSpec((1, bn), lambda j, i: (0, j)))
        args += [lnw.reshape(1, n), lnb.reshape(1, n)]
    return pl.pallas_call(
        functools.partial(_mm_kernel, act=act, ln=ln, res=has_res,
                          out_dtype=out_dtype),
        grid=(nn, nm),
        in_specs=in_specs,
        out_specs=pl.BlockSpec((bm, bn), lambda j, i: (i, j)),
        out_shape=jax.ShapeDtypeStruct((m, n), out_dtype),
        compiler_params=pltpu.CompilerParams(
            dimension_semantics=("parallel", "parallel")),
    )(*args)


def _attn_kernel(qkv_ref, o_ref):
    scale = 1.0 / math.sqrt(HD)
    tb = qkv_ref.shape[1]
    qb = [qkv_ref[i, :, 0:H].astype(_F32) for i in range(B)]
    kb = [qkv_ref[i, :, H:2 * H].astype(_F32) for i in range(B)]
    vb = [qkv_ref[i, :, 2 * H:3 * H].astype(_F32) for i in range(B)]
    for i in range(B):
        s = [(qb[i] * kb[j]).reshape(tb, NHEAD, HD).sum(axis=-1) * scale
             for j in range(B)]
        mx = jnp.maximum(jnp.maximum(s[0], s[1]), jnp.maximum(s[2], s[3]))
        e = [jnp.exp(s[j] - mx) for j in range(B)]
        den = e[0] + e[1] + e[2] + e[3]
        acc = jnp.zeros((tb, H), _F32)
        for j in range(B):
            aj = (e[j] / den).astype(_BF).astype(_F32)
            ab = jnp.broadcast_to(aj[:, :, None], (tb, NHEAD, HD)
                                  ).reshape(tb, H)
            acc = acc + (ab * vb[j]).astype(_BF).astype(_F32)
        o_ref[i, :, :] = acc.astype(_BF)


def _attn(qkv3):
    tb = 128
    return pl.pallas_call(
        _attn_kernel,
        grid=(NP // tb,),
        in_specs=[
            pl.BlockSpec((B, tb, KQKV), lambda t: (0, t, 0)),
        ],
        out_specs=pl.BlockSpec((B, tb, H), lambda t: (0, t, 0)),
        out_shape=jax.ShapeDtypeStruct((B, NP, H), _BF),
        compiler_params=pltpu.CompilerParams(
            dimension_semantics=("arbitrary",)),
    )(qkv3)


def _rlv_kernel(h_ref, r2_ref, b_ref, vc_ref, o_ref, g_ref, *, nm):
    i = pl.program_id(0)

    @pl.when(i == 0)
    def _():
        g_ref[...] = jnp.zeros_like(g_ref)

    r = jax.lax.dot_general(
        h_ref[...].astype(_BF), r2_ref[...].astype(_BF),
        (((1,), (1,)), ((), ())), preferred_element_type=_F32)
    rd = (r + b_ref[...]).astype(_BF).astype(_F32)
    g_ref[...] += jnp.sum(rd * vc_ref[...], axis=0, keepdims=True)

    @pl.when(i == nm - 1)
    def _():
        o_ref[...] = g_ref[...]


def _rlv(hr, r2p, r2b, vcol, bm=1280):
    nm = MP // bm
    return pl.pallas_call(
        functools.partial(_rlv_kernel, nm=nm),
        grid=(nm,),
        in_specs=[
            pl.BlockSpec((bm, H), lambda i: (i, 0)),
            pl.BlockSpec((NP, H), lambda i: (0, 0)),
            pl.BlockSpec((1, NP), lambda i: (0, 0)),
            pl.BlockSpec((bm, 1), lambda i: (i, 0)),
        ],
        out_specs=pl.BlockSpec((1, NP), lambda i: (0, 0)),
        out_shape=jax.ShapeDtypeStruct((1, NP), _F32),
        scratch_shapes=[pltpu.VMEM((1, NP), _F32)],
        compiler_params=pltpu.CompilerParams(
            dimension_semantics=("arbitrary",)),
    )(hr, r2p, r2b.reshape(1, NP), vcol)


def _route_kernel(rlv_ref, nw_ref, nb_ref, rl_ref, sw_ref):
    rlv = rlv_ref[...]
    t_iota = jax.lax.broadcasted_iota(jnp.int32, (1, NP), 1)
    mask = t_iota < NE
    cnt = float(NE)
    mean = jnp.sum(jnp.where(mask, rlv, 0.0)) / cnt
    d = jnp.where(mask, rlv - mean, 0.0)
    var = jnp.sum(d * d) / cnt
    rln = d / jnp.sqrt(var + 1e-5) * nw_ref[...] + nb_ref[...]
    rl_ref[...] = rln

    neg = jnp.float32(-jnp.inf)
    vrow = jnp.where(mask, rln, neg)
    eye = (jax.lax.broadcasted_iota(jnp.int32, (NP, NP), 0) ==
           jax.lax.broadcasted_iota(jnp.int32, (NP, NP), 1)).astype(_F32)
    u_iota = jax.lax.broadcasted_iota(jnp.int32, (NP, 1), 0)
    vcolT = jax.lax.dot_general(eye, rln, (((1,), (1,)), ((), ())),
                                preferred_element_type=_F32,
                                precision=jax.lax.Precision.HIGHEST)
    vcol = jnp.where(u_iota < NE, vcolT, neg)
    lt = (jax.lax.broadcasted_iota(jnp.int32, (NP, NP), 0) <
          jax.lax.broadcasted_iota(jnp.int32, (NP, NP), 1))
    gt = (vcol > vrow).astype(_F32)
    eqlt = ((vcol == vrow) & lt).astype(_F32)
    rank = jnp.sum(gt + eqlt, axis=0, keepdims=True)
    sel = ((rank < float(TOPK)) & mask).astype(_F32)
    selcol = jax.lax.dot_general(eye, sel, (((1,), (1,)), ((), ())),
                                 preferred_element_type=_F32,
                                 precision=jax.lax.Precision.HIGHEST)
    cume = jnp.sum(selcol * lt.astype(_F32), axis=0, keepdims=True)
    vsel = jnp.where(sel > 0.5, rln, 0.0)
    w = vsel / jnp.sum(vsel)
    jrow = jax.lax.broadcasted_iota(jnp.int32, (SELP, NP), 0)
    cume_i = cume.astype(jnp.int32)
    sw_ref[...] = jnp.where(jrow == cume_i, 1.0, 0.0) * w


def _route(rlv, nwp, nbp):
    return pl.pallas_call(
        _route_kernel,
        in_specs=[pl.BlockSpec((1, NP), lambda: (0, 0)),
                  pl.BlockSpec((1, NP), lambda: (0, 0)),
                  pl.BlockSpec((1, NP), lambda: (0, 0))],
        out_specs=[pl.BlockSpec((1, NP), lambda: (0, 0)),
                   pl.BlockSpec((SELP, NP), lambda: (0, 0))],
        out_shape=[jax.ShapeDtypeStruct((1, NP), _F32),
                   jax.ShapeDtypeStruct((SELP, NP), _F32)],
    )(rlv, nwp, nbp)


def _final_kernel(sw_ref, x_ref, o_ref):
    sw = sw_ref[...]
    x = x_ref[0]
    swh = sw.astype(_BF)
    swl = (sw - swh.astype(_F32)).astype(_BF)
    xh = x.astype(_BF)
    xl = (x - xh.astype(_F32)).astype(_BF)

    def d(a, b):
        return jax.lax.dot_general(a, b, (((1,), (0,)), ((), ())),
                                   preferred_element_type=_F32)

    o_ref[0] = d(swh, xh) + (d(swh, xl) + d(swl, xh))


def _final(sw, x2r):
    return pl.pallas_call(
        _final_kernel,
        grid=(B,),
        in_specs=[pl.BlockSpec((SELP, NP), lambda b: (0, 0)),
                  pl.BlockSpec((1, NP, H), lambda b: (b, 0, 0))],
        out_specs=pl.BlockSpec((1, SELP, H), lambda b: (b, 0, 0)),
        out_shape=jax.ShapeDtypeStruct((B, SELP, H), _F32),
        compiler_params=pltpu.CompilerParams(
            dimension_semantics=("arbitrary",)),
    )(sw, x2r)


def kernel(hidden_states, text_hidden_states, label_hidden_states,
           label_mask, params):
    p = params
    x = jnp.concatenate([hidden_states, text_hidden_states], axis=1)
    x = jnp.pad(x, ((0, 0), (0, NP - NTOK), (0, 0)))
    xf = x.reshape(MP, H)

    qkv = _mm(xf, p['in_proj_w'], p['in_proj_b'],
              bm=512, out_dtype=_BF)
    o = _attn(qkv.reshape(B, NP, KQKV))
    x1 = _mm(o.reshape(MP, H), p['out_proj_w'], p['out_proj_b'],
             res=xf, lnw=p['ln1_w'], lnb=p['ln1_b'], bm=640)
    h1 = _mm(x1, p['l1_w'], p['l1_b'], act="relu",
             bm=640, out_dtype=_BF)
    x2 = _mm(h1, p['l2_w'], p['l2_b'],
             res=x1, lnw=p['ln2_w'], lnb=p['ln2_b'], bm=640)
    hr = _mm(x2, p['r1_w'], p['r1_b'], act="gelu",
             bm=640, out_dtype=_BF)

    voter_bf = p['voter'][:, 0].astype(_BF).astype(_F32)
    vp = jnp.pad(voter_bf, (0, NP - NTOK)) * 0.25
    vcol = jnp.tile(vp, (B,)).reshape(MP, 1)

    r2p = jnp.pad(p['r2_w'], ((0, NP - NE), (0, 0)))
    r2b = jnp.pad(p['r2_b'], (0, NP - NE))
    rlv = _rlv(hr, r2p, r2b, vcol)

    nwp = jnp.pad(p['norm_w'].reshape(1, NE), ((0, 0), (0, NP - NE)))
    nbp = jnp.pad(p['norm_b'].reshape(1, NE), ((0, 0), (0, NP - NE)))
    rl_p, sw = _route(rlv, nwp, nbp)

    fin = _final(sw, x2.reshape(B, NP, H))
    return fin[:, :TOPK, :], rl_p[:, :NE]
